# Initial kernel scaffold; baseline (speedup 1.0000x reference)
import jax
import jax.numpy as jnp
from jax import lax
from jax.experimental import pallas as pl
from jax.experimental.pallas import tpu as pltpu
from jax.experimental.pallas import tpu_sc as plsc

N = 1_000_000
B = 16_384
NC = 2
NS = 16
L = 16
NW = NC * NS
CHUNK = 31_264
LAST = N - (NW - 1) * CHUNK
VECS = B // L
B_W = B // NW
G_I = 128
G_K = B_W // G_I


def _scatter_max_body(lu_hbm, ts_hbm, ids_hbm, out_hbm, tbl, ids_v, ts_v):
    wid = lax.axis_index("s") * NC + lax.axis_index("c")
    base = wid * CHUNK
    is_last = wid == NW - 1

    @pl.when(jnp.logical_not(is_last))
    def _():
        pltpu.sync_copy(lu_hbm.at[pl.ds(base, CHUNK)], tbl.at[pl.ds(0, CHUNK)])

    @pl.when(is_last)
    def _():
        pltpu.sync_copy(lu_hbm.at[pl.ds(base, LAST)], tbl.at[pl.ds(0, LAST)])

    pltpu.sync_copy(ids_hbm, ids_v)
    pltpu.sync_copy(ts_hbm, ts_v)

    hi = jnp.minimum(base + CHUNK, N)

    def step(i, carry):
        ids = ids_v[pl.ds(i * L, L)]
        ts = ts_v[pl.ds(i * L, L)]
        m = (ids >= base) & (ids < hi)
        li = jnp.where(m, ids - base, 0)

        def cond(need):
            return jnp.any(need)

        def body(need):
            cur = plsc.load_gather(tbl, [li], mask=need)
            upd = jnp.maximum(cur, ts)
            plsc.store_scatter(tbl, [li], upd, mask=need)
            cur2 = plsc.load_gather(tbl, [li], mask=need)
            return need & (cur2 < ts)

        lax.while_loop(cond, body, m)
        return carry

    lax.fori_loop(0, VECS, step, 0)

    @pl.when(jnp.logical_not(is_last))
    def _():
        pltpu.sync_copy(tbl.at[pl.ds(0, CHUNK)], out_hbm.at[pl.ds(base, CHUNK)])

    @pl.when(is_last)
    def _():
        pltpu.sync_copy(tbl.at[pl.ds(0, LAST)], out_hbm.at[pl.ds(base, LAST)])


def _gather_rel_body(upd_hbm, ts_hbm, ids2_hbm, rel_hbm, idx2, vals, ts_v,
                     rel_v, sem):
    wid = lax.axis_index("s") * NC + lax.axis_index("c")
    base = wid * B_W
    pltpu.sync_copy(ids2_hbm.at[pl.ds(wid * G_K, G_K)], idx2)
    pltpu.sync_copy(ts_hbm.at[pl.ds(base, B_W)], ts_v)
    copies = [
        pltpu.async_copy(upd_hbm.at[idx2.at[j]],
                         vals.at[pl.ds(j * G_I, G_I)], sem)
        for j in range(G_K)
    ]
    for c in copies:
        c.wait()
    for k in range(B_W // L):
        lu = vals[pl.ds(k * L, L)]
        t = ts_v[pl.ds(k * L, L)]
        rel_v[pl.ds(k * L, L)] = t - jnp.where(jnp.isinf(lu), t, lu)
    pltpu.sync_copy(rel_v, rel_hbm.at[pl.ds(base, B_W)])


def kernel(last_update, times, dst_ids):
    mesh = plsc.VectorSubcoreMesh(core_axis_name="c", subcore_axis_name="s")
    scatter_max = pl.kernel(
        _scatter_max_body,
        out_type=jax.ShapeDtypeStruct((N,), jnp.float32),
        mesh=mesh,
        scratch_types=[
            pltpu.VMEM((CHUNK,), jnp.float32),
            pltpu.VMEM((B,), jnp.int32),
            pltpu.VMEM((B,), jnp.float32),
        ],
    )
    gather_rel = pl.kernel(
        _gather_rel_body,
        out_type=jax.ShapeDtypeStruct((B,), jnp.float32),
        mesh=mesh,
        scratch_types=[
            pltpu.VMEM((G_K, G_I), jnp.int32),
            pltpu.VMEM((B_W,), jnp.float32),
            pltpu.VMEM((B_W,), jnp.float32),
            pltpu.VMEM((B_W,), jnp.float32),
            pltpu.SemaphoreType.DMA,
        ],
    )
    ids = dst_ids.astype(jnp.int32)
    updated = scatter_max(last_update, times, ids)
    rel = gather_rel(updated, times, ids.reshape(B // G_I, G_I))
    return (updated, rel)


# speedup vs baseline: 1.3586x; 1.3586x over previous
import jax
import jax.numpy as jnp
from jax import lax
from jax.experimental import pallas as pl
from jax.experimental.pallas import tpu as pltpu
from jax.experimental.pallas import tpu_sc as plsc

N = 1_000_000
B = 16_384
NC = 2
NS = 16
L = 16
NW = NC * NS
CHUNK = 31_264
LAST = N - (NW - 1) * CHUNK
VECS = B // L
B_W = B // NW
G_I = 128
G_K = B_W // G_I
SENT = 2_147_483_647

_DNUMS = lax.GatherDimensionNumbers(
    offset_dims=(), collapsed_slice_dims=(0,), start_index_map=(0,))


def _vgather(x, i):
    return lax.gather(x, i[:, None], _DNUMS, (1,),
                      mode=lax.GatherScatterMode.PROMISE_IN_BOUNDS)


def _scatter_max_body(lu_hbm, ts_hbm, ids_hbm, out_hbm, tbl, ids_v, ts_v):
    wid = lax.axis_index("s") * NC + lax.axis_index("c")
    base = wid * CHUNK
    is_last = wid == NW - 1

    @pl.when(jnp.logical_not(is_last))
    def _():
        pltpu.sync_copy(lu_hbm.at[pl.ds(base, CHUNK)], tbl.at[pl.ds(0, CHUNK)])

    @pl.when(is_last)
    def _():
        pltpu.sync_copy(lu_hbm.at[pl.ds(base, LAST)], tbl.at[pl.ds(0, LAST)])

    pltpu.sync_copy(ids_hbm, ids_v)
    pltpu.sync_copy(ts_hbm, ts_v)

    hi = jnp.minimum(base + CHUNK, N)

    def step(i, carry):
        ids = ids_v[pl.ds(i * L, L)]
        m = (ids >= base) & (ids < hi)
        cnt = plsc.all_reduce_population_count(m)[0]

        @pl.when(cnt > 0)
        def _():
            ts = ts_v[pl.ds(i * L, L)]
            li = jnp.where(m, ids - base, 0)
            cur = plsc.load_gather(tbl, [li], mask=m)
            upd = jnp.maximum(cur, ts)
            plsc.store_scatter(tbl, [li], upd, mask=m)

        @pl.when(cnt > 1)
        def _():
            ts = ts_v[pl.ds(i * L, L)]
            li = jnp.where(m, ids - base, 0)

            def rnd(j, rem):
                cur2 = plsc.load_gather(tbl, [li], mask=rem)
                rem2 = rem & (cur2 < ts)
                cur3 = plsc.load_gather(tbl, [li], mask=rem2)
                plsc.store_scatter(tbl, [li], jnp.maximum(cur3, ts), mask=rem2)
                return rem2

            lax.fori_loop(0, cnt - 1, rnd, m)

        return carry

    lax.fori_loop(0, VECS, step, 0)

    @pl.when(jnp.logical_not(is_last))
    def _():
        pltpu.sync_copy(tbl.at[pl.ds(0, CHUNK)], out_hbm.at[pl.ds(base, CHUNK)])

    @pl.when(is_last)
    def _():
        pltpu.sync_copy(tbl.at[pl.ds(0, LAST)], out_hbm.at[pl.ds(base, LAST)])


def _gather_rel_body(upd_hbm, ts_hbm, ids2_hbm, rel_hbm, idx2, vals, ts_v,
                     rel_v, sem):
    wid = lax.axis_index("s") * NC + lax.axis_index("c")
    base = wid * B_W
    pltpu.sync_copy(ids2_hbm.at[pl.ds(wid * G_K, G_K)], idx2)
    pltpu.sync_copy(ts_hbm.at[pl.ds(base, B_W)], ts_v)
    copies = [
        pltpu.async_copy(upd_hbm.at[idx2.at[j]],
                         vals.at[pl.ds(j * G_I, G_I)], sem)
        for j in range(G_K)
    ]
    for c in copies:
        c.wait()
    for k in range(B_W // L):
        lu = vals[pl.ds(k * L, L)]
        t = ts_v[pl.ds(k * L, L)]
        rel_v[pl.ds(k * L, L)] = t - jnp.where(jnp.isinf(lu), t, lu)
    pltpu.sync_copy(rel_v, rel_hbm.at[pl.ds(base, B_W)])


def kernel(last_update, times, dst_ids):
    mesh = plsc.VectorSubcoreMesh(core_axis_name="c", subcore_axis_name="s")
    scatter_max = pl.kernel(
        _scatter_max_body,
        out_type=jax.ShapeDtypeStruct((N,), jnp.float32),
        mesh=mesh,
        compiler_params=pltpu.CompilerParams(needs_layout_passes=False),
        scratch_types=[
            pltpu.VMEM((CHUNK,), jnp.float32),
            pltpu.VMEM((B,), jnp.int32),
            pltpu.VMEM((B,), jnp.float32),
        ],
    )
    gather_rel = pl.kernel(
        _gather_rel_body,
        out_type=jax.ShapeDtypeStruct((B,), jnp.float32),
        mesh=mesh,
        compiler_params=pltpu.CompilerParams(needs_layout_passes=False),
        scratch_types=[
            pltpu.VMEM((G_K, G_I), jnp.int32),
            pltpu.VMEM((B_W,), jnp.float32),
            pltpu.VMEM((B_W,), jnp.float32),
            pltpu.VMEM((B_W,), jnp.float32),
            pltpu.SemaphoreType.DMA,
        ],
    )
    ids = dst_ids.astype(jnp.int32)
    updated = scatter_max(last_update, times, ids)
    rel = gather_rel(updated, times, ids.reshape(B // G_I, G_I))
    return (updated, rel)


# speedup vs baseline: 2.1974x; 1.6173x over previous
import jax
import jax.numpy as jnp
from jax import lax
from jax.experimental import pallas as pl
from jax.experimental.pallas import tpu as pltpu
from jax.experimental.pallas import tpu_sc as plsc

N = 1_000_000
B = 16_384
NC = 2
NS = 16
L = 16
NW = NC * NS
CHUNK = 31_264
LAST = N - (NW - 1) * CHUNK
VECS = B // L
B_W = B // NW
G_I = 128
G_K = B_W // G_I
SENT = 2_147_483_647

_DNUMS = lax.GatherDimensionNumbers(
    offset_dims=(), collapsed_slice_dims=(0,), start_index_map=(0,))


def _vgather(x, i):
    return lax.gather(x, i[:, None], _DNUMS, (1,),
                      mode=lax.GatherScatterMode.PROMISE_IN_BOUNDS)


def _scatter_max_body(lu_hbm, ts_hbm, ids_hbm, out_hbm, tbl, ids_v, ts_v,
                      cids, cts, dsem):
    wid = lax.axis_index("s") * NC + lax.axis_index("c")
    base = wid * CHUNK
    is_last = wid == NW - 1
    hi = jnp.minimum(base + CHUNK, N)

    @pl.when(jnp.logical_not(is_last))
    def _():
        pltpu.async_copy(lu_hbm.at[pl.ds(base, CHUNK)],
                         tbl.at[pl.ds(0, CHUNK)], dsem)

    @pl.when(is_last)
    def _():
        pltpu.async_copy(lu_hbm.at[pl.ds(base, LAST)],
                         tbl.at[pl.ds(0, LAST)], dsem)

    pltpu.sync_copy(ids_hbm, ids_v)
    pltpu.sync_copy(ts_hbm, ts_v)

    iota = lax.iota(jnp.int32, L)

    @plsc.parallel_loop(0, VECS, carry=jnp.int32(0))
    def scan(i, pos):
        ids = ids_v[pl.ds(i * L, L)]
        ts = ts_v[pl.ds(i * L, L)]
        m = (ids >= base) & (ids < hi)
        cnt = plsc.all_reduce_population_count(m)[0]
        s_id, s_t = plsc.sort_key_val(jnp.where(m, ids, SENT), ts)
        keep = iota < cnt
        plsc.store_scatter(cids, [pos + iota], s_id, mask=keep)
        plsc.store_scatter(cts, [pos + iota], s_t, mask=keep)
        return pos + cnt

    W = scan

    @pl.when(jnp.logical_not(is_last))
    def _():
        pltpu.make_async_copy(lu_hbm.at[pl.ds(base, CHUNK)],
                              tbl.at[pl.ds(0, CHUNK)], dsem).wait()

    @pl.when(is_last)
    def _():
        pltpu.make_async_copy(lu_hbm.at[pl.ds(base, LAST)],
                              tbl.at[pl.ds(0, LAST)], dsem).wait()

    nv = (W + L - 1) >> 4

    def rmw(i, carry):
        ids = cids[pl.ds(i * L, L)]
        ts = cts[pl.ds(i * L, L)]
        valid = (i * L + iota) < W
        s_id, s_t = plsc.sort_key_val(jnp.where(valid, ids, SENT),
                                      jnp.where(valid, ts, -jnp.inf))
        nxt = _vgather(s_id, jnp.minimum(iota + 1, L - 1))
        final = ((iota == L - 1) | (s_id != nxt)) & (s_id != SENT)
        mv = s_t
        for d in (1, 2, 4, 8):
            src = jnp.maximum(iota - d, 0)
            pv = _vgather(mv, src)
            pid = _vgather(s_id, src)
            mv = jnp.where((iota >= d) & (pid == s_id),
                           jnp.maximum(mv, pv), mv)
        li = jnp.where(final, s_id - base, 0)
        cur = plsc.load_gather(tbl, [li], mask=final)
        plsc.store_scatter(tbl, [li], jnp.maximum(cur, mv), mask=final)
        return carry

    lax.fori_loop(0, nv, rmw, 0)

    @pl.when(jnp.logical_not(is_last))
    def _():
        pltpu.sync_copy(tbl.at[pl.ds(0, CHUNK)], out_hbm.at[pl.ds(base, CHUNK)])

    @pl.when(is_last)
    def _():
        pltpu.sync_copy(tbl.at[pl.ds(0, LAST)], out_hbm.at[pl.ds(base, LAST)])


def _gather_rel_body(upd_hbm, ts_hbm, ids2_hbm, rel_hbm, idx2, vals, ts_v,
                     rel_v, sem):
    wid = lax.axis_index("s") * NC + lax.axis_index("c")
    base = wid * B_W
    pltpu.sync_copy(ids2_hbm.at[pl.ds(wid * G_K, G_K)], idx2)
    pltpu.sync_copy(ts_hbm.at[pl.ds(base, B_W)], ts_v)
    copies = [
        pltpu.async_copy(upd_hbm.at[idx2.at[j]],
                         vals.at[pl.ds(j * G_I, G_I)], sem)
        for j in range(G_K)
    ]
    for c in copies:
        c.wait()
    for k in range(B_W // L):
        lu = vals[pl.ds(k * L, L)]
        t = ts_v[pl.ds(k * L, L)]
        rel_v[pl.ds(k * L, L)] = t - jnp.where(jnp.isinf(lu), t, lu)
    pltpu.sync_copy(rel_v, rel_hbm.at[pl.ds(base, B_W)])


def kernel(last_update, times, dst_ids):
    mesh = plsc.VectorSubcoreMesh(core_axis_name="c", subcore_axis_name="s")
    scatter_max = pl.kernel(
        _scatter_max_body,
        out_type=jax.ShapeDtypeStruct((N,), jnp.float32),
        mesh=mesh,
        compiler_params=pltpu.CompilerParams(needs_layout_passes=False),
        scratch_types=[
            pltpu.VMEM((CHUNK,), jnp.float32),
            pltpu.VMEM((B,), jnp.int32),
            pltpu.VMEM((B,), jnp.float32),
            pltpu.VMEM((B,), jnp.int32),
            pltpu.VMEM((B,), jnp.float32),
            pltpu.SemaphoreType.DMA,
        ],
    )
    gather_rel = pl.kernel(
        _gather_rel_body,
        out_type=jax.ShapeDtypeStruct((B,), jnp.float32),
        mesh=mesh,
        compiler_params=pltpu.CompilerParams(needs_layout_passes=False),
        scratch_types=[
            pltpu.VMEM((G_K, G_I), jnp.int32),
            pltpu.VMEM((B_W,), jnp.float32),
            pltpu.VMEM((B_W,), jnp.float32),
            pltpu.VMEM((B_W,), jnp.float32),
            pltpu.SemaphoreType.DMA,
        ],
    )
    ids = dst_ids.astype(jnp.int32)
    updated = scatter_max(last_update, times, ids)
    rel = gather_rel(updated, times, ids.reshape(B // G_I, G_I))
    return (updated, rel)
